# baseline (speedup 1.0000x reference)
import functools

import jax
import jax.numpy as jnp
from jax import lax
from jax.experimental import pallas as pl
from jax.experimental.pallas import tpu as pltpu
from jax.experimental.pallas import tpu_sc as plsc

N_NODES = 100000
D_FEAT = 128
CHUNK = 80
NBUF = 3
NMAIN = 39
BIG = NMAIN * CHUNK + 8
SMALL = NMAIN * CHUNK
NBIG = 20
TAIL = 8

_INFO = plsc.get_sparse_core_info()
_NC = _INFO.num_cores
_NS = _INFO.num_subcores
_NW = _NC * _NS


def _sc_body(x_hbm, idx_hbm, tab_hbm, out_hbm, *scratch):
    idx_all = scratch[0]
    rows_v = scratch[1:1 + NBUF]
    x_v = scratch[1 + NBUF:1 + 2 * NBUF]
    sem_g = scratch[1 + 2 * NBUF:1 + 3 * NBUF]
    sem_x = scratch[1 + 3 * NBUF:1 + 4 * NBUF]
    sem_o = scratch[1 + 4 * NBUF:1 + 5 * NBUF]

    wid = lax.axis_index("s") * _NC + lax.axis_index("c")
    span = wid * SMALL + jnp.minimum(wid, NBIG) * TAIL

    @pl.when(wid < NBIG)
    def _():
        pltpu.sync_copy(idx_hbm.at[pl.ds(span, BIG)], idx_all)

    @pl.when(wid >= NBIG)
    def _():
        pltpu.sync_copy(idx_hbm.at[pl.ds(span, SMALL)],
                        idx_all.at[pl.ds(0, SMALL)])

    def load(k, b):
        base = span + k * CHUNK
        pltpu.async_copy(tab_hbm.at[idx_all.at[pl.ds(k * CHUNK, CHUNK)]],
                         rows_v[b], sem_g[b])
        pltpu.async_copy(x_hbm.at[pl.ds(base, CHUNK), :], x_v[b], sem_x[b])

    def wait_loads(b):
        pltpu.make_async_copy(
            tab_hbm.at[idx_all.at[pl.ds(0, CHUNK)]], rows_v[b],
            sem_g[b]).wait()
        pltpu.make_async_copy(x_hbm.at[pl.ds(0, CHUNK), :], x_v[b],
                              sem_x[b]).wait()

    def add_rows(b):
        @plsc.parallel_loop(0, CHUNK, 1, unroll=4)
        def _(r):
            for c in range(D_FEAT // 16):
                sl = pl.ds(c * 16, 16)
                plsc.addupdate(x_v[b].at[r, sl], rows_v[b][r, sl])

    def store(k, b):
        base = span + k * CHUNK
        pltpu.async_copy(x_v[b], out_hbm.at[pl.ds(base, CHUNK), :], sem_o[b])

    def wait_store(b):
        pltpu.make_async_copy(x_v[b], out_hbm.at[pl.ds(0, CHUNK), :],
                              sem_o[b]).wait()

    load(0, 0)

    def turn(j, carry):
        for b in range(NBUF):
            k = j * NBUF + b
            bn = (b + 1) % NBUF
            @pl.when(k + 1 < NMAIN)
            def _():
                @pl.when(k >= 2)
                def _():
                    wait_store(bn)
                load(k + 1, bn)
            wait_loads(b)
            add_rows(b)
            store(k, b)
        return carry

    lax.fori_loop(0, NMAIN // NBUF, turn, 0, unroll=False)

    for b in range(NBUF):
        wait_store(b)

    @pl.when(wid < NBIG)
    def _():
        base = span + SMALL
        pltpu.async_copy(tab_hbm.at[idx_all.at[pl.ds(SMALL, TAIL)]],
                         rows_v[0].at[pl.ds(0, TAIL), :], sem_g[0])
        pltpu.async_copy(x_hbm.at[pl.ds(base, TAIL), :],
                         x_v[0].at[pl.ds(0, TAIL), :], sem_x[0])
        pltpu.make_async_copy(tab_hbm.at[idx_all.at[pl.ds(0, TAIL)]],
                              rows_v[0].at[pl.ds(0, TAIL), :], sem_g[0]).wait()
        pltpu.make_async_copy(x_hbm.at[pl.ds(0, TAIL), :],
                              x_v[0].at[pl.ds(0, TAIL), :], sem_x[0]).wait()
        for r in range(TAIL):
            for c in range(D_FEAT // 16):
                sl = pl.ds(c * 16, 16)
                plsc.addupdate(x_v[0].at[r, sl], rows_v[0][r, sl])
        pltpu.sync_copy(x_v[0].at[pl.ds(0, TAIL), :],
                        out_hbm.at[pl.ds(base, TAIL), :])


@jax.jit
def _run(x, idx, tab):
    mesh = plsc.VectorSubcoreMesh(core_axis_name="c", subcore_axis_name="s")
    f = pl.kernel(
        _sc_body,
        out_type=jax.ShapeDtypeStruct((N_NODES, D_FEAT), jnp.float32),
        mesh=mesh,
        scratch_types=(
            [pltpu.VMEM((BIG,), jnp.int32)]
            + [pltpu.VMEM((CHUNK, D_FEAT), jnp.float32) for _ in range(NBUF)]
            + [pltpu.VMEM((CHUNK, D_FEAT), jnp.float32) for _ in range(NBUF)]
            + [pltpu.SemaphoreType.DMA for _ in range(3 * NBUF)]
        ),
    )
    return f(x, idx, tab)


def kernel(x, node_types, type_table):
    idx = node_types.astype(jnp.int32)
    return _run(x, idx, type_table)


# speedup vs baseline: 1.8080x; 1.8080x over previous
import functools

import jax
import jax.numpy as jnp
from jax import lax
from jax.experimental import pallas as pl
from jax.experimental.pallas import tpu as pltpu
from jax.experimental.pallas import tpu_sc as plsc

N_NODES = 100000
D_FEAT = 128
NUM_TYPES = 64
CHUNK = 80
NBUF = 3
NMAIN = 39
BIG = NMAIN * CHUNK + 8
SMALL = NMAIN * CHUNK
NBIG = 20
TAIL = 8

_INFO = plsc.get_sparse_core_info()
_NC = _INFO.num_cores
_NS = _INFO.num_subcores
_NW = _NC * _NS


def _sc_body(x_hbm, idx_hbm, tab_hbm, out_hbm, *scratch):
    tab_v = scratch[0]
    idx_all = scratch[1]
    x_v = scratch[2:2 + NBUF]
    sem_x = scratch[2 + NBUF:2 + 2 * NBUF]
    sem_o = scratch[2 + 2 * NBUF:2 + 3 * NBUF]

    wid = lax.axis_index("s") * _NC + lax.axis_index("c")
    span = wid * SMALL + jnp.minimum(wid, NBIG) * TAIL

    pltpu.sync_copy(tab_hbm, tab_v)

    @pl.when(wid < NBIG)
    def _():
        pltpu.sync_copy(idx_hbm.at[pl.ds(span, BIG)],
                        idx_all.at[pl.ds(0, BIG)])

    @pl.when(wid >= NBIG)
    def _():
        pltpu.sync_copy(idx_hbm.at[pl.ds(span, SMALL)],
                        idx_all.at[pl.ds(0, SMALL)])

    def load(k, b):
        base = span + k * CHUNK
        pltpu.async_copy(x_hbm.at[pl.ds(base, CHUNK), :], x_v[b], sem_x[b])

    def wait_loads(b):
        pltpu.make_async_copy(x_hbm.at[pl.ds(0, CHUNK), :], x_v[b],
                              sem_x[b]).wait()

    def add_rows(b, k):
        @plsc.parallel_loop(0, CHUNK // 16, 1, unroll=1)
        def _(g):
            t_vec = idx_all[pl.ds(k * CHUNK + g * 16, 16)]
            for j in range(16):
                t = t_vec[j]
                r = g * 16 + j
                for c in range(D_FEAT // 16):
                    sl = pl.ds(c * 16, 16)
                    plsc.addupdate(x_v[b].at[r, sl], tab_v[t, sl])

    def store(k, b):
        base = span + k * CHUNK
        pltpu.async_copy(x_v[b], out_hbm.at[pl.ds(base, CHUNK), :], sem_o[b])

    def wait_store(b):
        pltpu.make_async_copy(x_v[b], out_hbm.at[pl.ds(0, CHUNK), :],
                              sem_o[b]).wait()

    load(0, 0)

    def turn(j, carry):
        for b in range(NBUF):
            k = j * NBUF + b
            bn = (b + 1) % NBUF
            @pl.when(k + 1 < NMAIN)
            def _():
                @pl.when(k >= 2)
                def _():
                    wait_store(bn)
                load(k + 1, bn)
            wait_loads(b)
            add_rows(b, k)
            store(k, b)
        return carry

    lax.fori_loop(0, NMAIN // NBUF, turn, 0, unroll=False)

    for b in range(NBUF):
        wait_store(b)

    @pl.when(wid < NBIG)
    def _():
        base = span + SMALL
        pltpu.async_copy(x_hbm.at[pl.ds(base, TAIL), :],
                         x_v[0].at[pl.ds(0, TAIL), :], sem_x[0])
        pltpu.make_async_copy(x_hbm.at[pl.ds(0, TAIL), :],
                              x_v[0].at[pl.ds(0, TAIL), :], sem_x[0]).wait()
        t_vec = idx_all[pl.ds(SMALL, 16)]
        for r in range(TAIL):
            t = t_vec[r]
            for c in range(D_FEAT // 16):
                sl = pl.ds(c * 16, 16)
                plsc.addupdate(x_v[0].at[r, sl], tab_v[t, sl])
        pltpu.sync_copy(x_v[0].at[pl.ds(0, TAIL), :],
                        out_hbm.at[pl.ds(base, TAIL), :])


@jax.jit
def _run(x, idx, tab):
    mesh = plsc.VectorSubcoreMesh(core_axis_name="c", subcore_axis_name="s")
    f = pl.kernel(
        _sc_body,
        out_type=jax.ShapeDtypeStruct((N_NODES, D_FEAT), jnp.float32),
        mesh=mesh,
        scratch_types=(
            [pltpu.VMEM((NUM_TYPES, D_FEAT), jnp.float32)]
            + [pltpu.VMEM((BIG + 8, ), jnp.int32)]
            + [pltpu.VMEM((CHUNK, D_FEAT), jnp.float32) for _ in range(NBUF)]
            + [pltpu.SemaphoreType.DMA for _ in range(2 * NBUF)]
        ),
    )
    return f(x, idx, tab)


def kernel(x, node_types, type_table):
    idx = node_types.astype(jnp.int32)
    return _run(x, idx, type_table)


# speedup vs baseline: 3.0040x; 1.6616x over previous
import functools

import jax
import jax.numpy as jnp
from jax import lax
from jax.experimental import pallas as pl
from jax.experimental.pallas import tpu as pltpu
from jax.experimental.pallas import tpu_sc as plsc

N_NODES = 100000
D_FEAT = 128
NUM_TYPES = 64
CHUNK = 80
NBUF = 3
NMAIN = 39
BIG = NMAIN * CHUNK + 8
SMALL = NMAIN * CHUNK
NBIG = 20
TAIL = 8

_INFO = plsc.get_sparse_core_info()
_NC = _INFO.num_cores
_NS = _INFO.num_subcores
_NW = _NC * _NS


def _sc_body(x_hbm, idx_hbm, tab_hbm, out_hbm, *scratch):
    tab_v = scratch[0]
    idx_all = scratch[1]
    x_v = scratch[2:2 + NBUF]
    sem_x = scratch[2 + NBUF:2 + 2 * NBUF]
    sem_o = scratch[2 + 2 * NBUF:2 + 3 * NBUF]

    wid = lax.axis_index("s") * _NC + lax.axis_index("c")
    span = wid * SMALL + jnp.minimum(wid, NBIG) * TAIL

    @pl.when(lax.axis_index("s") == 0)
    def _():
        pltpu.sync_copy(tab_hbm, tab_v)
    plsc.subcore_barrier()

    @pl.when(wid < NBIG)
    def _():
        pltpu.sync_copy(idx_hbm.at[pl.ds(span, BIG)],
                        idx_all.at[pl.ds(0, BIG)])

    @pl.when(wid >= NBIG)
    def _():
        pltpu.sync_copy(idx_hbm.at[pl.ds(span, SMALL)],
                        idx_all.at[pl.ds(0, SMALL)])

    def load(k, b):
        base = span + k * CHUNK
        pltpu.async_copy(x_hbm.at[pl.ds(base, CHUNK), :], x_v[b], sem_x[b])

    def wait_loads(b):
        pltpu.make_async_copy(x_hbm.at[pl.ds(0, CHUNK), :], x_v[b],
                              sem_x[b]).wait()

    def add_rows(b, k):
        pltpu.sync_copy(tab_v.at[idx_all.at[pl.ds(k * CHUNK, CHUNK)]],
                        x_v[b], add=True)

    def store(k, b):
        base = span + k * CHUNK
        pltpu.async_copy(x_v[b], out_hbm.at[pl.ds(base, CHUNK), :], sem_o[b])

    def wait_store(b):
        pltpu.make_async_copy(x_v[b], out_hbm.at[pl.ds(0, CHUNK), :],
                              sem_o[b]).wait()

    load(0, 0)

    def turn(j, carry):
        for b in range(NBUF):
            k = j * NBUF + b
            bn = (b + 1) % NBUF
            @pl.when(k + 1 < NMAIN)
            def _():
                @pl.when(k >= 2)
                def _():
                    wait_store(bn)
                load(k + 1, bn)
            wait_loads(b)
            add_rows(b, k)
            store(k, b)
        return carry

    lax.fori_loop(0, NMAIN // NBUF, turn, 0, unroll=False)

    for b in range(NBUF):
        wait_store(b)

    @pl.when(wid < NBIG)
    def _():
        base = span + SMALL
        pltpu.async_copy(x_hbm.at[pl.ds(base, TAIL), :],
                         x_v[0].at[pl.ds(0, TAIL), :], sem_x[0])
        pltpu.make_async_copy(x_hbm.at[pl.ds(0, TAIL), :],
                              x_v[0].at[pl.ds(0, TAIL), :], sem_x[0]).wait()
        pltpu.sync_copy(tab_v.at[idx_all.at[pl.ds(SMALL, TAIL)]],
                        x_v[0].at[pl.ds(0, TAIL), :], add=True)
        pltpu.sync_copy(x_v[0].at[pl.ds(0, TAIL), :],
                        out_hbm.at[pl.ds(base, TAIL), :])


@jax.jit
def _run(x, idx, tab):
    mesh = plsc.VectorSubcoreMesh(core_axis_name="c", subcore_axis_name="s")
    f = pl.kernel(
        _sc_body,
        out_type=jax.ShapeDtypeStruct((N_NODES, D_FEAT), jnp.float32),
        mesh=mesh,
        scratch_types=(
            [pltpu.VMEM_SHARED((NUM_TYPES, D_FEAT), jnp.float32)]
            + [pltpu.VMEM((BIG + 8, ), jnp.int32)]
            + [pltpu.VMEM((CHUNK, D_FEAT), jnp.float32) for _ in range(NBUF)]
            + [pltpu.SemaphoreType.DMA for _ in range(2 * NBUF)]
        ),
    )
    return f(x, idx, tab)


def kernel(x, node_types, type_table):
    idx = node_types.astype(jnp.int32)
    return _run(x, idx, type_table)


# speedup vs baseline: 3.2166x; 1.0708x over previous
import functools

import jax
import jax.numpy as jnp
from jax import lax
from jax.experimental import pallas as pl
from jax.experimental.pallas import tpu as pltpu
from jax.experimental.pallas import tpu_sc as plsc

N_NODES = 100000
D_FEAT = 128
NUM_TYPES = 64
CHUNK = 128
NBUF = 3
NMAIN = 24
BIG = NMAIN * CHUNK + 56
SMALL = NMAIN * CHUNK + 48
NBIG = 20
TAIL_B = 56
TAIL_S = 48

_INFO = plsc.get_sparse_core_info()
_NC = _INFO.num_cores
_NS = _INFO.num_subcores
_NW = _NC * _NS


def _sc_body(x_hbm, idx_hbm, tab_hbm, out_hbm, *scratch):
    tab_v = scratch[0]
    idx_all = scratch[1]
    x_v = scratch[2:2 + NBUF]
    sem_x = scratch[2 + NBUF:2 + 2 * NBUF]
    sem_o = scratch[2 + 2 * NBUF:2 + 3 * NBUF]

    wid = lax.axis_index("s") * _NC + lax.axis_index("c")
    span = wid * SMALL + jnp.minimum(wid, NBIG) * (TAIL_B - TAIL_S)

    @pl.when(lax.axis_index("s") == 0)
    def _():
        pltpu.sync_copy(tab_hbm, tab_v)
    plsc.subcore_barrier()

    @pl.when(wid < NBIG)
    def _():
        pltpu.sync_copy(idx_hbm.at[pl.ds(span, BIG)],
                        idx_all.at[pl.ds(0, BIG)])

    @pl.when(wid >= NBIG)
    def _():
        pltpu.sync_copy(idx_hbm.at[pl.ds(span, SMALL)],
                        idx_all.at[pl.ds(0, SMALL)])

    def load(k, b):
        base = span + k * CHUNK
        pltpu.async_copy(x_hbm.at[pl.ds(base, CHUNK), :], x_v[b], sem_x[b])

    def wait_loads(b):
        pltpu.make_async_copy(x_hbm.at[pl.ds(0, CHUNK), :], x_v[b],
                              sem_x[b]).wait()

    def add_rows(b, k):
        pltpu.sync_copy(tab_v.at[idx_all.at[pl.ds(k * CHUNK, CHUNK)]],
                        x_v[b], add=True)

    def store(k, b):
        base = span + k * CHUNK
        pltpu.async_copy(x_v[b], out_hbm.at[pl.ds(base, CHUNK), :], sem_o[b])

    def wait_store(b):
        pltpu.make_async_copy(x_v[b], out_hbm.at[pl.ds(0, CHUNK), :],
                              sem_o[b]).wait()

    load(0, 0)

    def turn(j, carry):
        for b in range(NBUF):
            k = j * NBUF + b
            bn = (b + 1) % NBUF
            @pl.when(k + 1 < NMAIN)
            def _():
                @pl.when(k >= 2)
                def _():
                    wait_store(bn)
                load(k + 1, bn)
            wait_loads(b)
            add_rows(b, k)
            store(k, b)
        return carry

    lax.fori_loop(0, NMAIN // NBUF, turn, 0, unroll=False)

    for b in range(NBUF):
        wait_store(b)

    def do_tail(nrows):
        base = span + NMAIN * CHUNK
        pltpu.async_copy(x_hbm.at[pl.ds(base, nrows), :],
                         x_v[0].at[pl.ds(0, nrows), :], sem_x[0])
        pltpu.make_async_copy(x_hbm.at[pl.ds(0, nrows), :],
                              x_v[0].at[pl.ds(0, nrows), :], sem_x[0]).wait()
        pltpu.sync_copy(tab_v.at[idx_all.at[pl.ds(NMAIN * CHUNK, nrows)]],
                        x_v[0].at[pl.ds(0, nrows), :], add=True)
        pltpu.sync_copy(x_v[0].at[pl.ds(0, nrows), :],
                        out_hbm.at[pl.ds(base, nrows), :])

    @pl.when(wid < NBIG)
    def _():
        do_tail(TAIL_B)

    @pl.when(wid >= NBIG)
    def _():
        do_tail(TAIL_S)


@jax.jit
def _run(x, idx, tab):
    mesh = plsc.VectorSubcoreMesh(core_axis_name="c", subcore_axis_name="s")
    f = pl.kernel(
        _sc_body,
        out_type=jax.ShapeDtypeStruct((N_NODES, D_FEAT), jnp.float32),
        mesh=mesh,
        scratch_types=(
            [pltpu.VMEM_SHARED((NUM_TYPES, D_FEAT), jnp.float32)]
            + [pltpu.VMEM((BIG,), jnp.int32)]
            + [pltpu.VMEM((CHUNK, D_FEAT), jnp.float32) for _ in range(NBUF)]
            + [pltpu.SemaphoreType.DMA for _ in range(2 * NBUF)]
        ),
    )
    return f(x, idx, tab)


def kernel(x, node_types, type_table):
    idx = node_types.astype(jnp.int32)
    return _run(x, idx, type_table)


# speedup vs baseline: 3.2231x; 1.0020x over previous
import functools

import jax
import jax.numpy as jnp
from jax import lax
from jax.experimental import pallas as pl
from jax.experimental.pallas import tpu as pltpu
from jax.experimental.pallas import tpu_sc as plsc

N_NODES = 100000
D_FEAT = 128
NUM_TYPES = 64
CHUNK = 128
NBUF = 4
NMAIN = 24
BIG = NMAIN * CHUNK + 56
SMALL = NMAIN * CHUNK + 48
NBIG = 20
TAIL_B = 56
TAIL_S = 48

_INFO = plsc.get_sparse_core_info()
_NC = _INFO.num_cores
_NS = _INFO.num_subcores
_NW = _NC * _NS


def _sc_body(x_hbm, idx_hbm, tab_hbm, out_hbm, *scratch):
    tab_v = scratch[0]
    idx_all = scratch[1]
    x_v = scratch[2:2 + NBUF]
    sem_x = scratch[2 + NBUF:2 + 2 * NBUF]
    sem_o = scratch[2 + 2 * NBUF:2 + 3 * NBUF]

    wid = lax.axis_index("s") * _NC + lax.axis_index("c")
    span = wid * SMALL + jnp.minimum(wid, NBIG) * (TAIL_B - TAIL_S)

    @pl.when(lax.axis_index("s") == 0)
    def _():
        pltpu.sync_copy(tab_hbm, tab_v)
    plsc.subcore_barrier()

    @pl.when(wid < NBIG)
    def _():
        pltpu.sync_copy(idx_hbm.at[pl.ds(span, BIG)],
                        idx_all.at[pl.ds(0, BIG)])

    @pl.when(wid >= NBIG)
    def _():
        pltpu.sync_copy(idx_hbm.at[pl.ds(span, SMALL)],
                        idx_all.at[pl.ds(0, SMALL)])

    def load(k, b):
        base = span + k * CHUNK
        pltpu.async_copy(x_hbm.at[pl.ds(base, CHUNK), :], x_v[b], sem_x[b])

    def wait_loads(b):
        pltpu.make_async_copy(x_hbm.at[pl.ds(0, CHUNK), :], x_v[b],
                              sem_x[b]).wait()

    def add_rows(b, k):
        pltpu.sync_copy(tab_v.at[idx_all.at[pl.ds(k * CHUNK, CHUNK)]],
                        x_v[b], add=True)

    def store(k, b):
        base = span + k * CHUNK
        pltpu.async_copy(x_v[b], out_hbm.at[pl.ds(base, CHUNK), :], sem_o[b])

    def wait_store(b):
        pltpu.make_async_copy(x_v[b], out_hbm.at[pl.ds(0, CHUNK), :],
                              sem_o[b]).wait()

    load(0, 0)

    def turn(j, carry):
        for b in range(NBUF):
            k = j * NBUF + b
            bn = (b + 1) % NBUF
            @pl.when(k + 1 < NMAIN)
            def _():
                @pl.when(k >= NBUF - 1)
                def _():
                    wait_store(bn)
                load(k + 1, bn)
            wait_loads(b)
            add_rows(b, k)
            store(k, b)
        return carry

    lax.fori_loop(0, NMAIN // NBUF, turn, 0, unroll=False)

    wait_store(0)
    tail_base = span + NMAIN * CHUNK

    @pl.when(wid < NBIG)
    def _():
        pltpu.async_copy(x_hbm.at[pl.ds(tail_base, TAIL_B), :],
                         x_v[0].at[pl.ds(0, TAIL_B), :], sem_x[0])

    @pl.when(wid >= NBIG)
    def _():
        pltpu.async_copy(x_hbm.at[pl.ds(tail_base, TAIL_S), :],
                         x_v[0].at[pl.ds(0, TAIL_S), :], sem_x[0])

    for b in range(1, NBUF):
        wait_store(b)

    def do_tail(nrows):
        base = span + NMAIN * CHUNK
        pltpu.make_async_copy(x_hbm.at[pl.ds(0, nrows), :],
                              x_v[0].at[pl.ds(0, nrows), :], sem_x[0]).wait()
        pltpu.sync_copy(tab_v.at[idx_all.at[pl.ds(NMAIN * CHUNK, nrows)]],
                        x_v[0].at[pl.ds(0, nrows), :], add=True)
        pltpu.sync_copy(x_v[0].at[pl.ds(0, nrows), :],
                        out_hbm.at[pl.ds(base, nrows), :])

    @pl.when(wid < NBIG)
    def _():
        do_tail(TAIL_B)

    @pl.when(wid >= NBIG)
    def _():
        do_tail(TAIL_S)


@jax.jit
def _run(x, idx, tab):
    mesh = plsc.VectorSubcoreMesh(core_axis_name="c", subcore_axis_name="s")
    f = pl.kernel(
        _sc_body,
        out_type=jax.ShapeDtypeStruct((N_NODES, D_FEAT), jnp.float32),
        mesh=mesh,
        scratch_types=(
            [pltpu.VMEM_SHARED((NUM_TYPES, D_FEAT), jnp.float32)]
            + [pltpu.VMEM((BIG,), jnp.int32)]
            + [pltpu.VMEM((CHUNK, D_FEAT), jnp.float32) for _ in range(NBUF)]
            + [pltpu.SemaphoreType.DMA for _ in range(2 * NBUF)]
        ),
    )
    return f(x, idx, tab)


def kernel(x, node_types, type_table):
    idx = node_types.astype(jnp.int32)
    return _run(x, idx, type_table)
